# Initial kernel scaffold; baseline (speedup 1.0000x reference)
import jax
import jax.numpy as jnp
from jax.experimental import pallas as pl


def kernel(pos, x, W_head, b_head, g_head, be_head, Wb, bb, gb, beb, Wf, bf, gf, bef, W1, b1, g1, be1, W2, b2, g2, be2, W3, b3, batch):
    raise NotImplementedError("write your pallas kernel here")


# speedup vs baseline: 1.0083x; 1.0083x over previous
import functools
from typing import Any

import jax
import jax.numpy as jnp
from jax import lax
from jax.experimental import pallas as pl
from jax.experimental.pallas import tpu as pltpu

N = 8192
B = 4
NPC = 2048
K = 16
NB = 7
C = 64
INC = 9
NCLS = 13
FUSE = C * NB
EPS = 1e-5


_RB = 512


def _mm_stats_body(x_ref, w_ref, b_ref, z_ref, st_ref, acc_ref):
    i = pl.program_id(0)
    z = jnp.dot(x_ref[...], w_ref[...], preferred_element_type=jnp.float32)
    z = z + b_ref[...]
    z_ref[...] = z

    @pl.when(i == 0)
    def _():
        acc_ref[...] = jnp.zeros_like(acc_ref)

    acc_ref[0, :] += jnp.sum(z, axis=0)
    acc_ref[1, :] += jnp.sum(z * z, axis=0)

    @pl.when(i == pl.num_programs(0) - 1)
    def _():
        st_ref[...] = acc_ref[...]


def _mm_stats(x, w, b):
    n, din = x.shape
    dout = w.shape[1]
    grid = n // _RB
    z, st = pl.pallas_call(
        _mm_stats_body,
        grid=(grid,),
        in_specs=[
            pl.BlockSpec((_RB, din), lambda i: (i, 0)),
            pl.BlockSpec((din, dout), lambda i: (0, 0)),
            pl.BlockSpec((dout,), lambda i: (0,)),
        ],
        out_specs=[
            pl.BlockSpec((_RB, dout), lambda i: (i, 0)),
            pl.BlockSpec((2, dout), lambda i: (0, 0)),
        ],
        out_shape=[
            jax.ShapeDtypeStruct((n, dout), jnp.float32),
            jax.ShapeDtypeStruct((2, dout), jnp.float32),
        ],
        scratch_shapes=[pltpu.VMEM((2, dout), jnp.float32)],
    )(x, w, b)
    return z, st


def _bn_coeffs(st, n_rows, g, be):
    mean = st[0] / n_rows
    var = st[1] / n_rows - mean * mean
    inv = g * lax.rsqrt(var + EPS)
    return inv, be - mean * inv


def _segmax_body(z_ref, a_ref, c_ref, o_ref):
    i = pl.program_id(0)
    zn = jnp.maximum(z_ref[...] * a_ref[...] + c_ref[...], 0.0)
    o_ref[pl.ds(i, 1), :] = jnp.max(zn, axis=0, keepdims=True)


def _bn_relu_segmax(z, st, g, be):
    n, dout = z.shape
    a, c = _bn_coeffs(st, n, g, be)
    a = a.reshape(1, dout)
    c = c.reshape(1, dout)
    return pl.pallas_call(
        _segmax_body,
        grid=(B,),
        in_specs=[
            pl.BlockSpec((NPC, dout), lambda i: (i, 0)),
            pl.BlockSpec((1, dout), lambda i: (0, 0)),
            pl.BlockSpec((1, dout), lambda i: (0, 0)),
        ],
        out_specs=pl.BlockSpec((B, dout), lambda i: (0, 0)),
        out_shape=jax.ShapeDtypeStruct((B, dout), jnp.float32),
    )(z, a, c)


def _fuse1_body(f_ref, fs_ref, wt_ref, wb_ref, b_ref, z_ref, st_ref, acc_ref):
    i = pl.program_id(0)
    j = pl.program_id(1)
    z = jnp.dot(f_ref[...], wb_ref[...], preferred_element_type=jnp.float32)
    fs = fs_ref[pl.ds(i, 1), :]
    z = z + jnp.dot(fs, wt_ref[...], preferred_element_type=jnp.float32)
    z = z + b_ref[...]
    z_ref[...] = z

    @pl.when(jnp.logical_and(i == 0, j == 0))
    def _():
        acc_ref[...] = jnp.zeros_like(acc_ref)

    acc_ref[0, :] += jnp.sum(z, axis=0)
    acc_ref[1, :] += jnp.sum(z * z, axis=0)

    @pl.when(jnp.logical_and(i == pl.num_programs(0) - 1,
                             j == pl.num_programs(1) - 1))
    def _():
        st_ref[...] = acc_ref[...]


def _fuse1(F, fu_seg, W1, b1):
    dtop = fu_seg.shape[1]
    dout = W1.shape[1]
    W1t = W1[:dtop]
    W1b = W1[dtop:]
    z, st = pl.pallas_call(
        _fuse1_body,
        grid=(B, NPC // _RB),
        in_specs=[
            pl.BlockSpec((_RB, FUSE), lambda i, j: (i * (NPC // _RB) + j, 0)),
            pl.BlockSpec((B, dtop), lambda i, j: (0, 0)),
            pl.BlockSpec((dtop, dout), lambda i, j: (0, 0)),
            pl.BlockSpec((FUSE, dout), lambda i, j: (0, 0)),
            pl.BlockSpec((dout,), lambda i, j: (0,)),
        ],
        out_specs=[
            pl.BlockSpec((_RB, dout), lambda i, j: (i * (NPC // _RB) + j, 0)),
            pl.BlockSpec((2, dout), lambda i, j: (0, 0)),
        ],
        out_shape=[
            jax.ShapeDtypeStruct((N, dout), jnp.float32),
            jax.ShapeDtypeStruct((2, dout), jnp.float32),
        ],
        scratch_shapes=[pltpu.VMEM((2, dout), jnp.float32)],
    )(F, fu_seg, W1t, W1b, b1)
    return z, st


def _act_mm_body(z_ref, a_ref, c_ref, w_ref, b_ref, o_ref, st_ref, acc_ref):
    i = pl.program_id(0)
    zn = jnp.maximum(z_ref[...] * a_ref[...] + c_ref[...], 0.0)
    o = jnp.dot(zn, w_ref[...], preferred_element_type=jnp.float32) + b_ref[...]
    o_ref[...] = o

    @pl.when(i == 0)
    def _():
        acc_ref[...] = jnp.zeros_like(acc_ref)

    acc_ref[0, :] += jnp.sum(o, axis=0)
    acc_ref[1, :] += jnp.sum(o * o, axis=0)

    @pl.when(i == pl.num_programs(0) - 1)
    def _():
        st_ref[...] = acc_ref[...]


def _act_mm(z, st, g, be, w, b):
    n, din = z.shape
    dout = w.shape[1]
    a, c = _bn_coeffs(st, n, g, be)
    a = a.reshape(1, din)
    c = c.reshape(1, din)
    o, st2 = pl.pallas_call(
        _act_mm_body,
        grid=(n // _RB,),
        in_specs=[
            pl.BlockSpec((_RB, din), lambda i: (i, 0)),
            pl.BlockSpec((1, din), lambda i: (0, 0)),
            pl.BlockSpec((1, din), lambda i: (0, 0)),
            pl.BlockSpec((din, dout), lambda i: (0, 0)),
            pl.BlockSpec((dout,), lambda i: (0,)),
        ],
        out_specs=[
            pl.BlockSpec((_RB, dout), lambda i: (i, 0)),
            pl.BlockSpec((2, dout), lambda i: (0, 0)),
        ],
        out_shape=[
            jax.ShapeDtypeStruct((n, dout), jnp.float32),
            jax.ShapeDtypeStruct((2, dout), jnp.float32),
        ],
        scratch_shapes=[pltpu.VMEM((2, dout), jnp.float32)],
    )(z, a, c, w, b)
    return o, st2


def _knn(f, kk):
    fb = f.reshape(B, NPC, -1)
    sq = jnp.sum(fb * fb, axis=-1)
    dist = sq[:, :, None] + sq[:, None, :] - 2.0 * jnp.einsum(
        'bnd,bmd->bnm', fb, fb)
    dist = jnp.where(jnp.eye(NPC, dtype=bool)[None, :, :], jnp.inf, dist)
    idx = lax.top_k(-dist, kk)[1]
    idx = idx + (jnp.arange(B) * NPC)[:, None, None]
    return idx.reshape(N, kk)


def _edge_layer(x, idx, W, b, g, be, res):
    kk = idx.shape[1]
    xi = jnp.broadcast_to(x[:, None, :], (N, kk, x.shape[1]))
    xj = x[idx]
    e = jnp.concatenate([xi, xj - xi], axis=-1).reshape(N * kk, -1)
    ez = e @ W + b
    m_ = jnp.mean(ez, axis=0)
    v_ = jnp.var(ez, axis=0)
    en = jax.nn.relu((ez - m_) / jnp.sqrt(v_ + 1e-5) * g + be)
    out = jnp.max(en.reshape(N, kk, -1), axis=1)
    if res is not None:
        out = out + res
    return out


def kernel(pos, x, W_head, b_head, g_head, be_head, Wb, bb, gb, beb,
           Wf, bf, gf, bef, W1, b1, g1, be1, W2, b2, g2, be2, W3, b3, batch):
    h0 = jnp.concatenate([pos, x], axis=1)
    idx0 = _knn(h0[:, 0:3], K)
    feats = [_edge_layer(h0, idx0, W_head, b_head, g_head, be_head, None)]
    for i in range(NB - 1):
        d = i + 1
        idx = _knn(feats[-1], K * d)[:, ::d]
        feats.append(_edge_layer(feats[-1], idx, Wb[i], bb[i], gb[i], beb[i],
                                 feats[-1]))
    F = jnp.concatenate(feats, axis=1)

    zf, stf = _mm_stats(F, Wf, bf)
    fu_seg = _bn_relu_segmax(zf, stf, gf, bef)
    z1, st1 = _fuse1(F, fu_seg, W1, b1)
    z2, st2 = _act_mm(z1, st1, g1, be1, W2, b2)
    out, _ = _act_mm(z2, st2, g2, be2, W3, b3)
    return out


# speedup vs baseline: 1.5932x; 1.5801x over previous
import functools
from typing import Any

import jax
import jax.numpy as jnp
from jax import lax
from jax.experimental import pallas as pl
from jax.experimental.pallas import tpu as pltpu

N = 8192
B = 4
NPC = 2048
K = 16
NB = 7
C = 64
INC = 9
NCLS = 13
FUSE = C * NB
EPS = 1e-5


_RB = 512


def _mm_stats_body(x_ref, w_ref, b_ref, z_ref, st_ref, acc_ref):
    i = pl.program_id(0)
    z = jnp.dot(x_ref[...], w_ref[...], preferred_element_type=jnp.float32)
    z = z + b_ref[...]
    z_ref[...] = z

    @pl.when(i == 0)
    def _():
        acc_ref[...] = jnp.zeros_like(acc_ref)

    acc_ref[0, :] += jnp.sum(z, axis=0)
    acc_ref[1, :] += jnp.sum(z * z, axis=0)

    @pl.when(i == pl.num_programs(0) - 1)
    def _():
        st_ref[...] = acc_ref[...]


def _mm_stats(x, w, b):
    n, din = x.shape
    dout = w.shape[1]
    grid = n // _RB
    z, st = pl.pallas_call(
        _mm_stats_body,
        grid=(grid,),
        in_specs=[
            pl.BlockSpec((_RB, din), lambda i: (i, 0)),
            pl.BlockSpec((din, dout), lambda i: (0, 0)),
            pl.BlockSpec((dout,), lambda i: (0,)),
        ],
        out_specs=[
            pl.BlockSpec((_RB, dout), lambda i: (i, 0)),
            pl.BlockSpec((2, dout), lambda i: (0, 0)),
        ],
        out_shape=[
            jax.ShapeDtypeStruct((n, dout), jnp.float32),
            jax.ShapeDtypeStruct((2, dout), jnp.float32),
        ],
        scratch_shapes=[pltpu.VMEM((2, dout), jnp.float32)],
    )(x, w, b)
    return z, st


def _bn_coeffs(st, n_rows, g, be):
    mean = st[0] / n_rows
    var = st[1] / n_rows - mean * mean
    inv = g * lax.rsqrt(var + EPS)
    return inv, be - mean * inv


def _segmax_body(z_ref, a_ref, c_ref, o_ref):
    i = pl.program_id(0)
    zn = jnp.maximum(z_ref[...] * a_ref[...] + c_ref[...], 0.0)
    o_ref[pl.ds(i, 1), :] = jnp.max(zn, axis=0, keepdims=True)


def _bn_relu_segmax(z, st, g, be):
    n, dout = z.shape
    a, c = _bn_coeffs(st, n, g, be)
    a = a.reshape(1, dout)
    c = c.reshape(1, dout)
    return pl.pallas_call(
        _segmax_body,
        grid=(B,),
        in_specs=[
            pl.BlockSpec((NPC, dout), lambda i: (i, 0)),
            pl.BlockSpec((1, dout), lambda i: (0, 0)),
            pl.BlockSpec((1, dout), lambda i: (0, 0)),
        ],
        out_specs=pl.BlockSpec((B, dout), lambda i: (0, 0)),
        out_shape=jax.ShapeDtypeStruct((B, dout), jnp.float32),
    )(z, a, c)


def _fuse1_body(f_ref, fs_ref, wt_ref, wb_ref, b_ref, z_ref, st_ref, acc_ref):
    i = pl.program_id(0)
    j = pl.program_id(1)
    z = jnp.dot(f_ref[...], wb_ref[...], preferred_element_type=jnp.float32)
    fs = fs_ref[pl.ds(i, 1), :]
    z = z + jnp.dot(fs, wt_ref[...], preferred_element_type=jnp.float32)
    z = z + b_ref[...]
    z_ref[...] = z

    @pl.when(jnp.logical_and(i == 0, j == 0))
    def _():
        acc_ref[...] = jnp.zeros_like(acc_ref)

    acc_ref[0, :] += jnp.sum(z, axis=0)
    acc_ref[1, :] += jnp.sum(z * z, axis=0)

    @pl.when(jnp.logical_and(i == pl.num_programs(0) - 1,
                             j == pl.num_programs(1) - 1))
    def _():
        st_ref[...] = acc_ref[...]


def _fuse1(F, fu_seg, W1, b1):
    dtop = fu_seg.shape[1]
    dout = W1.shape[1]
    W1t = W1[:dtop]
    W1b = W1[dtop:]
    z, st = pl.pallas_call(
        _fuse1_body,
        grid=(B, NPC // _RB),
        in_specs=[
            pl.BlockSpec((_RB, FUSE), lambda i, j: (i * (NPC // _RB) + j, 0)),
            pl.BlockSpec((B, dtop), lambda i, j: (0, 0)),
            pl.BlockSpec((dtop, dout), lambda i, j: (0, 0)),
            pl.BlockSpec((FUSE, dout), lambda i, j: (0, 0)),
            pl.BlockSpec((dout,), lambda i, j: (0,)),
        ],
        out_specs=[
            pl.BlockSpec((_RB, dout), lambda i, j: (i * (NPC // _RB) + j, 0)),
            pl.BlockSpec((2, dout), lambda i, j: (0, 0)),
        ],
        out_shape=[
            jax.ShapeDtypeStruct((N, dout), jnp.float32),
            jax.ShapeDtypeStruct((2, dout), jnp.float32),
        ],
        scratch_shapes=[pltpu.VMEM((2, dout), jnp.float32)],
    )(F, fu_seg, W1t, W1b, b1)
    return z, st


def _act_mm_body(z_ref, a_ref, c_ref, w_ref, b_ref, o_ref, st_ref, acc_ref):
    i = pl.program_id(0)
    zn = jnp.maximum(z_ref[...] * a_ref[...] + c_ref[...], 0.0)
    o = jnp.dot(zn, w_ref[...], preferred_element_type=jnp.float32) + b_ref[...]
    o_ref[...] = o

    @pl.when(i == 0)
    def _():
        acc_ref[...] = jnp.zeros_like(acc_ref)

    acc_ref[0, :] += jnp.sum(o, axis=0)
    acc_ref[1, :] += jnp.sum(o * o, axis=0)

    @pl.when(i == pl.num_programs(0) - 1)
    def _():
        st_ref[...] = acc_ref[...]


def _act_mm(z, st, g, be, w, b):
    n, din = z.shape
    dout = w.shape[1]
    a, c = _bn_coeffs(st, n, g, be)
    a = a.reshape(1, din)
    c = c.reshape(1, din)
    o, st2 = pl.pallas_call(
        _act_mm_body,
        grid=(n // _RB,),
        in_specs=[
            pl.BlockSpec((_RB, din), lambda i: (i, 0)),
            pl.BlockSpec((1, din), lambda i: (0, 0)),
            pl.BlockSpec((1, din), lambda i: (0, 0)),
            pl.BlockSpec((din, dout), lambda i: (0, 0)),
            pl.BlockSpec((dout,), lambda i: (0,)),
        ],
        out_specs=[
            pl.BlockSpec((_RB, dout), lambda i: (i, 0)),
            pl.BlockSpec((2, dout), lambda i: (0, 0)),
        ],
        out_shape=[
            jax.ShapeDtypeStruct((n, dout), jnp.float32),
            jax.ShapeDtypeStruct((2, dout), jnp.float32),
        ],
        scratch_shapes=[pltpu.VMEM((2, dout), jnp.float32)],
    )(z, a, c, w, b)
    return o, st2


def _knn(f, kk):
    fb = f.reshape(B, NPC, -1)
    sq = jnp.sum(fb * fb, axis=-1)
    dist = sq[:, :, None] + sq[:, None, :] - 2.0 * jnp.einsum(
        'bnd,bmd->bnm', fb, fb)
    dist = jnp.where(jnp.eye(NPC, dtype=bool)[None, :, :], jnp.inf, dist)
    idx = lax.top_k(-dist, kk)[1]
    idx = idx + (jnp.arange(B) * NPC)[:, None, None]
    return idx.reshape(N, kk)


_KROWS = 256


def _knn_body(kk, dil, fr_ref, fc_ref, sqr_ref, sqc_ref, o_ref,
              dist_ref, idx_ref):
    r = pl.program_id(1)
    g = jax.lax.dot_general(fr_ref[...], fc_ref[...], (((1,), (1,)), ((), ())),
                            preferred_element_type=jnp.float32)
    t1 = sqr_ref[...].reshape(_KROWS, 1) + sqc_ref[...].reshape(1, NPC)
    dist = t1 - 2.0 * g
    colv = lax.broadcasted_iota(jnp.int32, (_KROWS, NPC), 1)
    rowv = lax.broadcasted_iota(jnp.int32, (_KROWS, NPC), 0) + r * _KROWS
    dist = jnp.where(colv == rowv, jnp.inf, dist)
    dist_ref[...] = dist

    def step(t, _):
        dcur = dist_ref[...]
        vmin = jnp.min(dcur, axis=1, keepdims=True)
        tie = jnp.where(dcur == vmin, colv, NPC)
        amin = jnp.min(tie, axis=1)
        idx_ref[pl.ds(t, 1), :] = amin.reshape(1, _KROWS)
        dist_ref[...] = jnp.where(colv == amin[:, None], jnp.inf, dcur)
        return 0

    lax.fori_loop(0, kk, step, 0)
    o_ref[...] = jnp.concatenate(
        [idx_ref[i * dil:i * dil + 1, :] for i in range(K)], axis=0)


def _knn_pallas(f, kk, dil):
    fb = f.reshape(B, NPC, C)
    sq = jnp.sum(fb * fb, axis=-1).reshape(N)
    body = functools.partial(_knn_body, kk, dil)
    out = pl.pallas_call(
        body,
        grid=(B, NPC // _KROWS),
        in_specs=[
            pl.BlockSpec((_KROWS, C), lambda b, r: (b * (NPC // _KROWS) + r, 0)),
            pl.BlockSpec((NPC, C), lambda b, r: (b, 0)),
            pl.BlockSpec((_KROWS,), lambda b, r: (b * (NPC // _KROWS) + r,)),
            pl.BlockSpec((NPC,), lambda b, r: (b,)),
        ],
        out_specs=pl.BlockSpec((K, _KROWS),
                               lambda b, r: (0, b * (NPC // _KROWS) + r)),
        out_shape=jax.ShapeDtypeStruct((K, N), jnp.int32),
        scratch_shapes=[pltpu.VMEM((_KROWS, NPC), jnp.float32),
                        pltpu.VMEM((kk, _KROWS), jnp.int32)],
    )(f, f, sq, sq)
    base = jnp.repeat(jnp.arange(B, dtype=jnp.int32), NPC) * NPC
    return out.T + base[:, None]


def _edge_layer(x, idx, W, b, g, be, res):
    kk = idx.shape[1]
    xi = jnp.broadcast_to(x[:, None, :], (N, kk, x.shape[1]))
    xj = x[idx]
    e = jnp.concatenate([xi, xj - xi], axis=-1).reshape(N * kk, -1)
    ez = e @ W + b
    m_ = jnp.mean(ez, axis=0)
    v_ = jnp.var(ez, axis=0)
    en = jax.nn.relu((ez - m_) / jnp.sqrt(v_ + 1e-5) * g + be)
    out = jnp.max(en.reshape(N, kk, -1), axis=1)
    if res is not None:
        out = out + res
    return out


def kernel(pos, x, W_head, b_head, g_head, be_head, Wb, bb, gb, beb,
           Wf, bf, gf, bef, W1, b1, g1, be1, W2, b2, g2, be2, W3, b3, batch):
    h0 = jnp.concatenate([pos, x], axis=1)
    idx0 = _knn(h0[:, 0:3], K)
    feats = [_edge_layer(h0, idx0, W_head, b_head, g_head, be_head, None)]
    for i in range(NB - 1):
        d = i + 1
        idx = _knn_pallas(feats[-1], K * d, d)
        feats.append(_edge_layer(feats[-1], idx, Wb[i], bb[i], gb[i], beb[i],
                                 feats[-1]))
    F = jnp.concatenate(feats, axis=1)

    zf, stf = _mm_stats(F, Wf, bf)
    fu_seg = _bn_relu_segmax(zf, stf, gf, bef)
    z1, st1 = _fuse1(F, fu_seg, W1, b1)
    z2, st2 = _act_mm(z1, st1, g1, be1, W2, b2)
    out, _ = _act_mm(z2, st2, g2, be2, W3, b3)
    return out


# speedup vs baseline: 1.9436x; 1.2199x over previous
import functools
from typing import Any

import jax
import jax.numpy as jnp
from jax import lax
from jax.experimental import pallas as pl
from jax.experimental.pallas import tpu as pltpu

N = 8192
B = 4
NPC = 2048
K = 16
NB = 7
C = 64
INC = 9
NCLS = 13
FUSE = C * NB
EPS = 1e-5


_RB = 512


def _mm_stats_body(x_ref, w_ref, b_ref, z_ref, st_ref, acc_ref):
    i = pl.program_id(0)
    z = jnp.dot(x_ref[...], w_ref[...], preferred_element_type=jnp.float32)
    z = z + b_ref[...]
    z_ref[...] = z

    @pl.when(i == 0)
    def _():
        acc_ref[...] = jnp.zeros_like(acc_ref)

    acc_ref[0, :] += jnp.sum(z, axis=0)
    acc_ref[1, :] += jnp.sum(z * z, axis=0)

    @pl.when(i == pl.num_programs(0) - 1)
    def _():
        st_ref[...] = acc_ref[...]


def _mm_stats(x, w, b):
    n, din = x.shape
    dout = w.shape[1]
    grid = n // _RB
    z, st = pl.pallas_call(
        _mm_stats_body,
        grid=(grid,),
        in_specs=[
            pl.BlockSpec((_RB, din), lambda i: (i, 0)),
            pl.BlockSpec((din, dout), lambda i: (0, 0)),
            pl.BlockSpec((dout,), lambda i: (0,)),
        ],
        out_specs=[
            pl.BlockSpec((_RB, dout), lambda i: (i, 0)),
            pl.BlockSpec((2, dout), lambda i: (0, 0)),
        ],
        out_shape=[
            jax.ShapeDtypeStruct((n, dout), jnp.float32),
            jax.ShapeDtypeStruct((2, dout), jnp.float32),
        ],
        scratch_shapes=[pltpu.VMEM((2, dout), jnp.float32)],
    )(x, w, b)
    return z, st


def _bn_coeffs(st, n_rows, g, be):
    mean = st[0] / n_rows
    var = st[1] / n_rows - mean * mean
    inv = g * lax.rsqrt(var + EPS)
    return inv, be - mean * inv


def _segmax_body(z_ref, a_ref, c_ref, o_ref):
    i = pl.program_id(0)
    zn = jnp.maximum(z_ref[...] * a_ref[...] + c_ref[...], 0.0)
    o_ref[pl.ds(i, 1), :] = jnp.max(zn, axis=0, keepdims=True)


def _bn_relu_segmax(z, st, g, be):
    n, dout = z.shape
    a, c = _bn_coeffs(st, n, g, be)
    a = a.reshape(1, dout)
    c = c.reshape(1, dout)
    return pl.pallas_call(
        _segmax_body,
        grid=(B,),
        in_specs=[
            pl.BlockSpec((NPC, dout), lambda i: (i, 0)),
            pl.BlockSpec((1, dout), lambda i: (0, 0)),
            pl.BlockSpec((1, dout), lambda i: (0, 0)),
        ],
        out_specs=pl.BlockSpec((B, dout), lambda i: (0, 0)),
        out_shape=jax.ShapeDtypeStruct((B, dout), jnp.float32),
    )(z, a, c)


def _fuse1_body(f_ref, fs_ref, wt_ref, wb_ref, b_ref, z_ref, st_ref, acc_ref):
    i = pl.program_id(0)
    j = pl.program_id(1)
    z = jnp.dot(f_ref[...], wb_ref[...], preferred_element_type=jnp.float32)
    fs = fs_ref[pl.ds(i, 1), :]
    z = z + jnp.dot(fs, wt_ref[...], preferred_element_type=jnp.float32)
    z = z + b_ref[...]
    z_ref[...] = z

    @pl.when(jnp.logical_and(i == 0, j == 0))
    def _():
        acc_ref[...] = jnp.zeros_like(acc_ref)

    acc_ref[0, :] += jnp.sum(z, axis=0)
    acc_ref[1, :] += jnp.sum(z * z, axis=0)

    @pl.when(jnp.logical_and(i == pl.num_programs(0) - 1,
                             j == pl.num_programs(1) - 1))
    def _():
        st_ref[...] = acc_ref[...]


def _fuse1(F, fu_seg, W1, b1):
    dtop = fu_seg.shape[1]
    dout = W1.shape[1]
    W1t = W1[:dtop]
    W1b = W1[dtop:]
    z, st = pl.pallas_call(
        _fuse1_body,
        grid=(B, NPC // _RB),
        in_specs=[
            pl.BlockSpec((_RB, FUSE), lambda i, j: (i * (NPC // _RB) + j, 0)),
            pl.BlockSpec((B, dtop), lambda i, j: (0, 0)),
            pl.BlockSpec((dtop, dout), lambda i, j: (0, 0)),
            pl.BlockSpec((FUSE, dout), lambda i, j: (0, 0)),
            pl.BlockSpec((dout,), lambda i, j: (0,)),
        ],
        out_specs=[
            pl.BlockSpec((_RB, dout), lambda i, j: (i * (NPC // _RB) + j, 0)),
            pl.BlockSpec((2, dout), lambda i, j: (0, 0)),
        ],
        out_shape=[
            jax.ShapeDtypeStruct((N, dout), jnp.float32),
            jax.ShapeDtypeStruct((2, dout), jnp.float32),
        ],
        scratch_shapes=[pltpu.VMEM((2, dout), jnp.float32)],
    )(F, fu_seg, W1t, W1b, b1)
    return z, st


def _act_mm_body(z_ref, a_ref, c_ref, w_ref, b_ref, o_ref, st_ref, acc_ref):
    i = pl.program_id(0)
    zn = jnp.maximum(z_ref[...] * a_ref[...] + c_ref[...], 0.0)
    o = jnp.dot(zn, w_ref[...], preferred_element_type=jnp.float32) + b_ref[...]
    o_ref[...] = o

    @pl.when(i == 0)
    def _():
        acc_ref[...] = jnp.zeros_like(acc_ref)

    acc_ref[0, :] += jnp.sum(o, axis=0)
    acc_ref[1, :] += jnp.sum(o * o, axis=0)

    @pl.when(i == pl.num_programs(0) - 1)
    def _():
        st_ref[...] = acc_ref[...]


def _act_mm(z, st, g, be, w, b):
    n, din = z.shape
    dout = w.shape[1]
    a, c = _bn_coeffs(st, n, g, be)
    a = a.reshape(1, din)
    c = c.reshape(1, din)
    o, st2 = pl.pallas_call(
        _act_mm_body,
        grid=(n // _RB,),
        in_specs=[
            pl.BlockSpec((_RB, din), lambda i: (i, 0)),
            pl.BlockSpec((1, din), lambda i: (0, 0)),
            pl.BlockSpec((1, din), lambda i: (0, 0)),
            pl.BlockSpec((din, dout), lambda i: (0, 0)),
            pl.BlockSpec((dout,), lambda i: (0,)),
        ],
        out_specs=[
            pl.BlockSpec((_RB, dout), lambda i: (i, 0)),
            pl.BlockSpec((2, dout), lambda i: (0, 0)),
        ],
        out_shape=[
            jax.ShapeDtypeStruct((n, dout), jnp.float32),
            jax.ShapeDtypeStruct((2, dout), jnp.float32),
        ],
        scratch_shapes=[pltpu.VMEM((2, dout), jnp.float32)],
    )(z, a, c, w, b)
    return o, st2


def _knn(f, kk):
    fb = f.reshape(B, NPC, -1)
    sq = jnp.sum(fb * fb, axis=-1)
    dist = sq[:, :, None] + sq[:, None, :] - 2.0 * jnp.einsum(
        'bnd,bmd->bnm', fb, fb)
    dist = jnp.where(jnp.eye(NPC, dtype=bool)[None, :, :], jnp.inf, dist)
    idx = lax.top_k(-dist, kk)[1]
    idx = idx + (jnp.arange(B) * NPC)[:, None, None]
    return idx.reshape(N, kk)


_KROWS = 256


def _knn_body(kk, dil, fr_ref, fc_ref, sqr_ref, sqc_ref, o_ref,
              dist_ref, idx_ref):
    r = pl.program_id(1)
    g = jax.lax.dot_general(fr_ref[...], fc_ref[...], (((1,), (1,)), ((), ())),
                            preferred_element_type=jnp.float32)
    t1 = sqr_ref[...].reshape(_KROWS, 1) + sqc_ref[...].reshape(1, NPC)
    dist = t1 - 2.0 * g
    colv = lax.broadcasted_iota(jnp.int32, (_KROWS, NPC), 1)
    rowv = lax.broadcasted_iota(jnp.int32, (_KROWS, NPC), 0) + r * _KROWS
    dist = jnp.where(colv == rowv, jnp.inf, dist)
    dist_ref[...] = dist

    def step(t, _):
        dcur = dist_ref[...]
        vmin = jnp.min(dcur, axis=1, keepdims=True)
        tie = jnp.where(dcur == vmin, colv, NPC)
        amin = jnp.min(tie, axis=1)
        idx_ref[pl.ds(t, 1), :] = amin.reshape(1, _KROWS)
        dist_ref[...] = jnp.where(colv == amin[:, None], jnp.inf, dcur)
        return 0

    lax.fori_loop(0, kk, step, 0)
    o_ref[...] = jnp.concatenate(
        [idx_ref[i * dil:i * dil + 1, :] for i in range(K)], axis=0)


def _knn_pallas(f, kk, dil):
    dim = f.shape[1]
    fb = f.reshape(B, NPC, dim)
    sq = jnp.sum(fb * fb, axis=-1).reshape(N)
    body = functools.partial(_knn_body, kk, dil)
    out = pl.pallas_call(
        body,
        grid=(B, NPC // _KROWS),
        in_specs=[
            pl.BlockSpec((_KROWS, dim), lambda b, r: (b * (NPC // _KROWS) + r, 0)),
            pl.BlockSpec((NPC, dim), lambda b, r: (b, 0)),
            pl.BlockSpec((_KROWS,), lambda b, r: (b * (NPC // _KROWS) + r,)),
            pl.BlockSpec((NPC,), lambda b, r: (b,)),
        ],
        out_specs=pl.BlockSpec((K, _KROWS),
                               lambda b, r: (0, b * (NPC // _KROWS) + r)),
        out_shape=jax.ShapeDtypeStruct((K, N), jnp.int32),
        scratch_shapes=[pltpu.VMEM((_KROWS, NPC), jnp.float32),
                        pltpu.VMEM((kk, _KROWS), jnp.int32)],
    )(f, f, sq, sq)
    base = jnp.repeat(jnp.arange(B, dtype=jnp.int32), NPC) * NPC
    return out.T + base[:, None]


def _edge_layer(x, idx, W, b, g, be, res):
    kk = idx.shape[1]
    xi = jnp.broadcast_to(x[:, None, :], (N, kk, x.shape[1]))
    xj = x[idx]
    e = jnp.concatenate([xi, xj - xi], axis=-1).reshape(N * kk, -1)
    ez = e @ W + b
    m_ = jnp.mean(ez, axis=0)
    v_ = jnp.var(ez, axis=0)
    en = jax.nn.relu((ez - m_) / jnp.sqrt(v_ + 1e-5) * g + be)
    out = jnp.max(en.reshape(N, kk, -1), axis=1)
    if res is not None:
        out = out + res
    return out


def kernel(pos, x, W_head, b_head, g_head, be_head, Wb, bb, gb, beb,
           Wf, bf, gf, bef, W1, b1, g1, be1, W2, b2, g2, be2, W3, b3, batch):
    h0 = jnp.concatenate([pos, x], axis=1)
    idx0 = _knn_pallas(h0[:, 0:3], K, 1)
    feats = [_edge_layer(h0, idx0, W_head, b_head, g_head, be_head, None)]
    for i in range(NB - 1):
        d = i + 1
        idx = _knn_pallas(feats[-1], 15 * d + 1, d)
        feats.append(_edge_layer(feats[-1], idx, Wb[i], bb[i], gb[i], beb[i],
                                 feats[-1]))
    F = jnp.concatenate(feats, axis=1)

    zf, stf = _mm_stats(F, Wf, bf)
    fu_seg = _bn_relu_segmax(zf, stf, gf, bef)
    z1, st1 = _fuse1(F, fu_seg, W1, b1)
    z2, st2 = _act_mm(z1, st1, g1, be1, W2, b2)
    out, _ = _act_mm(z2, st2, g2, be2, W3, b3)
    return out


# speedup vs baseline: 2.4088x; 1.2394x over previous
import functools
from typing import Any

import jax
import jax.numpy as jnp
from jax import lax
from jax.experimental import pallas as pl
from jax.experimental.pallas import tpu as pltpu
from jax.experimental.pallas import tpu_sc as plsc

N = 8192
B = 4
NPC = 2048
K = 16
NB = 7
C = 64
INC = 9
NCLS = 13
FUSE = C * NB
EPS = 1e-5


_RB = 512


def _mm_stats_body(x_ref, w_ref, b_ref, z_ref, st_ref, acc_ref):
    i = pl.program_id(0)
    z = jnp.dot(x_ref[...], w_ref[...], preferred_element_type=jnp.float32)
    z = z + b_ref[...]
    z_ref[...] = z

    @pl.when(i == 0)
    def _():
        acc_ref[...] = jnp.zeros_like(acc_ref)

    acc_ref[0, :] += jnp.sum(z, axis=0)
    acc_ref[1, :] += jnp.sum(z * z, axis=0)

    @pl.when(i == pl.num_programs(0) - 1)
    def _():
        st_ref[...] = acc_ref[...]


def _mm_stats(x, w, b):
    n, din = x.shape
    dout = w.shape[1]
    grid = n // _RB
    z, st = pl.pallas_call(
        _mm_stats_body,
        grid=(grid,),
        in_specs=[
            pl.BlockSpec((_RB, din), lambda i: (i, 0)),
            pl.BlockSpec((din, dout), lambda i: (0, 0)),
            pl.BlockSpec((dout,), lambda i: (0,)),
        ],
        out_specs=[
            pl.BlockSpec((_RB, dout), lambda i: (i, 0)),
            pl.BlockSpec((2, dout), lambda i: (0, 0)),
        ],
        out_shape=[
            jax.ShapeDtypeStruct((n, dout), jnp.float32),
            jax.ShapeDtypeStruct((2, dout), jnp.float32),
        ],
        scratch_shapes=[pltpu.VMEM((2, dout), jnp.float32)],
    )(x, w, b)
    return z, st


def _bn_coeffs(st, n_rows, g, be):
    mean = st[0] / n_rows
    var = st[1] / n_rows - mean * mean
    inv = g * lax.rsqrt(var + EPS)
    return inv, be - mean * inv


def _segmax_body(z_ref, a_ref, c_ref, o_ref):
    i = pl.program_id(0)
    zn = jnp.maximum(z_ref[...] * a_ref[...] + c_ref[...], 0.0)
    o_ref[pl.ds(i, 1), :] = jnp.max(zn, axis=0, keepdims=True)


def _bn_relu_segmax(z, st, g, be):
    n, dout = z.shape
    a, c = _bn_coeffs(st, n, g, be)
    a = a.reshape(1, dout)
    c = c.reshape(1, dout)
    return pl.pallas_call(
        _segmax_body,
        grid=(B,),
        in_specs=[
            pl.BlockSpec((NPC, dout), lambda i: (i, 0)),
            pl.BlockSpec((1, dout), lambda i: (0, 0)),
            pl.BlockSpec((1, dout), lambda i: (0, 0)),
        ],
        out_specs=pl.BlockSpec((B, dout), lambda i: (0, 0)),
        out_shape=jax.ShapeDtypeStruct((B, dout), jnp.float32),
    )(z, a, c)


def _fuse1_body(f_ref, fs_ref, wt_ref, wb_ref, b_ref, z_ref, st_ref, acc_ref):
    i = pl.program_id(0)
    j = pl.program_id(1)
    z = jnp.dot(f_ref[...], wb_ref[...], preferred_element_type=jnp.float32)
    fs = fs_ref[pl.ds(i, 1), :]
    z = z + jnp.dot(fs, wt_ref[...], preferred_element_type=jnp.float32)
    z = z + b_ref[...]
    z_ref[...] = z

    @pl.when(jnp.logical_and(i == 0, j == 0))
    def _():
        acc_ref[...] = jnp.zeros_like(acc_ref)

    acc_ref[0, :] += jnp.sum(z, axis=0)
    acc_ref[1, :] += jnp.sum(z * z, axis=0)

    @pl.when(jnp.logical_and(i == pl.num_programs(0) - 1,
                             j == pl.num_programs(1) - 1))
    def _():
        st_ref[...] = acc_ref[...]


def _fuse1(F, fu_seg, W1, b1):
    dtop = fu_seg.shape[1]
    dout = W1.shape[1]
    W1t = W1[:dtop]
    W1b = W1[dtop:]
    z, st = pl.pallas_call(
        _fuse1_body,
        grid=(B, NPC // _RB),
        in_specs=[
            pl.BlockSpec((_RB, FUSE), lambda i, j: (i * (NPC // _RB) + j, 0)),
            pl.BlockSpec((B, dtop), lambda i, j: (0, 0)),
            pl.BlockSpec((dtop, dout), lambda i, j: (0, 0)),
            pl.BlockSpec((FUSE, dout), lambda i, j: (0, 0)),
            pl.BlockSpec((dout,), lambda i, j: (0,)),
        ],
        out_specs=[
            pl.BlockSpec((_RB, dout), lambda i, j: (i * (NPC // _RB) + j, 0)),
            pl.BlockSpec((2, dout), lambda i, j: (0, 0)),
        ],
        out_shape=[
            jax.ShapeDtypeStruct((N, dout), jnp.float32),
            jax.ShapeDtypeStruct((2, dout), jnp.float32),
        ],
        scratch_shapes=[pltpu.VMEM((2, dout), jnp.float32)],
    )(F, fu_seg, W1t, W1b, b1)
    return z, st


def _act_mm_body(z_ref, a_ref, c_ref, w_ref, b_ref, o_ref, st_ref, acc_ref):
    i = pl.program_id(0)
    zn = jnp.maximum(z_ref[...] * a_ref[...] + c_ref[...], 0.0)
    o = jnp.dot(zn, w_ref[...], preferred_element_type=jnp.float32) + b_ref[...]
    o_ref[...] = o

    @pl.when(i == 0)
    def _():
        acc_ref[...] = jnp.zeros_like(acc_ref)

    acc_ref[0, :] += jnp.sum(o, axis=0)
    acc_ref[1, :] += jnp.sum(o * o, axis=0)

    @pl.when(i == pl.num_programs(0) - 1)
    def _():
        st_ref[...] = acc_ref[...]


def _act_mm(z, st, g, be, w, b):
    n, din = z.shape
    dout = w.shape[1]
    a, c = _bn_coeffs(st, n, g, be)
    a = a.reshape(1, din)
    c = c.reshape(1, din)
    o, st2 = pl.pallas_call(
        _act_mm_body,
        grid=(n // _RB,),
        in_specs=[
            pl.BlockSpec((_RB, din), lambda i: (i, 0)),
            pl.BlockSpec((1, din), lambda i: (0, 0)),
            pl.BlockSpec((1, din), lambda i: (0, 0)),
            pl.BlockSpec((din, dout), lambda i: (0, 0)),
            pl.BlockSpec((dout,), lambda i: (0,)),
        ],
        out_specs=[
            pl.BlockSpec((_RB, dout), lambda i: (i, 0)),
            pl.BlockSpec((2, dout), lambda i: (0, 0)),
        ],
        out_shape=[
            jax.ShapeDtypeStruct((n, dout), jnp.float32),
            jax.ShapeDtypeStruct((2, dout), jnp.float32),
        ],
        scratch_shapes=[pltpu.VMEM((2, dout), jnp.float32)],
    )(z, a, c, w, b)
    return o, st2


_SC_CHUNK = 512


def _sc_gather(table, idx_flat):
    n_idx = idx_flat.shape[0]
    dim = table.shape[1]
    info = plsc.get_sparse_core_info()
    nw = info.num_cores * info.num_subcores
    per_w = n_idx // nw
    steps = per_w // _SC_CHUNK
    mesh = plsc.VectorSubcoreMesh(core_axis_name="c", subcore_axis_name="s")

    @functools.partial(
        pl.kernel, mesh=mesh,
        out_type=jax.ShapeDtypeStruct((n_idx, dim), jnp.float32),
        scratch_types=[
            pltpu.VMEM((_SC_CHUNK,), jnp.int32),
            pltpu.VMEM((_SC_CHUNK, dim), jnp.float32),
            pltpu.SemaphoreType.DMA,
        ],
    )
    def k(table_hbm, idx_hbm, out_hbm, idx_v, rows_v, sem):
        wid = lax.axis_index("s") * info.num_cores + lax.axis_index("c")
        base = wid * per_w

        def step(i, carry):
            off = base + i * _SC_CHUNK
            pltpu.sync_copy(idx_hbm.at[pl.ds(off, _SC_CHUNK)], idx_v)
            pltpu.async_copy(table_hbm.at[idx_v], rows_v, sem).wait()
            pltpu.sync_copy(rows_v, out_hbm.at[pl.ds(off, _SC_CHUNK)])
            return carry

        lax.fori_loop(0, steps, step, 0)

    return k(table, idx_flat)


def _edge_gather(x, idx):
    dim = x.shape[1]
    kk = idx.shape[1]
    if dim != 128:
        xp = jnp.pad(x, ((0, 0), (0, 128 - dim)))
    else:
        xp = x
    rows = _sc_gather(xp, idx.reshape(-1))
    return rows[:, :dim].reshape(N, kk, dim)


def _knn(f, kk):
    fb = f.reshape(B, NPC, -1)
    sq = jnp.sum(fb * fb, axis=-1)
    dist = sq[:, :, None] + sq[:, None, :] - 2.0 * jnp.einsum(
        'bnd,bmd->bnm', fb, fb)
    dist = jnp.where(jnp.eye(NPC, dtype=bool)[None, :, :], jnp.inf, dist)
    idx = lax.top_k(-dist, kk)[1]
    idx = idx + (jnp.arange(B) * NPC)[:, None, None]
    return idx.reshape(N, kk)


_KROWS = 256


def _knn_body(kk, dil, fr_ref, fc_ref, sqr_ref, sqc_ref, o_ref,
              dist_ref, idx_ref):
    r = pl.program_id(1)
    g = jax.lax.dot_general(fr_ref[...], fc_ref[...], (((1,), (1,)), ((), ())),
                            preferred_element_type=jnp.float32)
    t1 = sqr_ref[...].reshape(_KROWS, 1) + sqc_ref[...].reshape(1, NPC)
    dist = t1 - 2.0 * g
    colv = lax.broadcasted_iota(jnp.int32, (_KROWS, NPC), 1)
    rowv = lax.broadcasted_iota(jnp.int32, (_KROWS, NPC), 0) + r * _KROWS
    dist = jnp.where(colv == rowv, jnp.inf, dist)
    dist_ref[...] = dist

    def step(t, _):
        dcur = dist_ref[...]
        vmin = jnp.min(dcur, axis=1, keepdims=True)
        tie = jnp.where(dcur == vmin, colv, NPC)
        amin = jnp.min(tie, axis=1)
        idx_ref[pl.ds(t, 1), :] = amin.reshape(1, _KROWS)
        dist_ref[...] = jnp.where(colv == amin[:, None], jnp.inf, dcur)
        return 0

    lax.fori_loop(0, kk, step, 0)
    o_ref[...] = jnp.concatenate(
        [idx_ref[i * dil:i * dil + 1, :] for i in range(K)], axis=0)


def _knn_pallas(f, kk, dil):
    dim = f.shape[1]
    fb = f.reshape(B, NPC, dim)
    sq = jnp.sum(fb * fb, axis=-1).reshape(N)
    body = functools.partial(_knn_body, kk, dil)
    out = pl.pallas_call(
        body,
        grid=(B, NPC // _KROWS),
        in_specs=[
            pl.BlockSpec((_KROWS, dim), lambda b, r: (b * (NPC // _KROWS) + r, 0)),
            pl.BlockSpec((NPC, dim), lambda b, r: (b, 0)),
            pl.BlockSpec((_KROWS,), lambda b, r: (b * (NPC // _KROWS) + r,)),
            pl.BlockSpec((NPC,), lambda b, r: (b,)),
        ],
        out_specs=pl.BlockSpec((K, _KROWS),
                               lambda b, r: (0, b * (NPC // _KROWS) + r)),
        out_shape=jax.ShapeDtypeStruct((K, N), jnp.int32),
        scratch_shapes=[pltpu.VMEM((_KROWS, NPC), jnp.float32),
                        pltpu.VMEM((kk, _KROWS), jnp.int32)],
    )(f, f, sq, sq)
    base = jnp.repeat(jnp.arange(B, dtype=jnp.int32), NPC) * NPC
    return out.T + base[:, None]


def _edge_layer(x, idx, W, b, g, be, res):
    kk = idx.shape[1]
    xi = jnp.broadcast_to(x[:, None, :], (N, kk, x.shape[1]))
    xj = _edge_gather(x, idx)
    e = jnp.concatenate([xi, xj - xi], axis=-1).reshape(N * kk, -1)
    ez = e @ W + b
    m_ = jnp.mean(ez, axis=0)
    v_ = jnp.var(ez, axis=0)
    en = jax.nn.relu((ez - m_) / jnp.sqrt(v_ + 1e-5) * g + be)
    out = jnp.max(en.reshape(N, kk, -1), axis=1)
    if res is not None:
        out = out + res
    return out


def kernel(pos, x, W_head, b_head, g_head, be_head, Wb, bb, gb, beb,
           Wf, bf, gf, bef, W1, b1, g1, be1, W2, b2, g2, be2, W3, b3, batch):
    h0 = jnp.concatenate([pos, x], axis=1)
    idx0 = _knn_pallas(h0[:, 0:3], K, 1)
    feats = [_edge_layer(h0, idx0, W_head, b_head, g_head, be_head, None)]
    for i in range(NB - 1):
        d = i + 1
        idx = _knn_pallas(feats[-1], 15 * d + 1, d)
        feats.append(_edge_layer(feats[-1], idx, Wb[i], bb[i], gb[i], beb[i],
                                 feats[-1]))
    F = jnp.concatenate(feats, axis=1)

    zf, stf = _mm_stats(F, Wf, bf)
    fu_seg = _bn_relu_segmax(zf, stf, gf, bef)
    z1, st1 = _fuse1(F, fu_seg, W1, b1)
    z2, st2 = _act_mm(z1, st1, g1, be1, W2, b2)
    out, _ = _act_mm(z2, st2, g2, be2, W3, b3)
    return out
